# baseline (speedup 1.0000x reference)
import functools

import jax
import jax.numpy as jnp
from jax import lax
from jax.experimental import pallas as pl
from jax.experimental.pallas import tpu as pltpu
from jax.experimental.pallas import tpu_sc as plsc

NC = 2
NS = 16
NW = NC * NS
LANES = 16
CH = 128
EPS = 1e-8


def _emb_body(ids_hbm, seg_hbm, word_hbm, pos_hbm, segemb_hbm, gam_hbm,
              bet_hbm, out_hbm, ids_v, segs_v, pos_v, g_v, b_v, bufA, bufB,
              semA, semB, *, n_chunks, L, D):
  wid = lax.axis_index("s") * NC + lax.axis_index("c")
  nk = D // LANES

  pltpu.sync_copy(ids_hbm.at[wid], ids_v)
  pltpu.sync_copy(seg_hbm.at[wid], segs_v)
  pltpu.sync_copy(pos_hbm, pos_v)
  pltpu.sync_copy(gam_hbm, g_v)
  pltpu.sync_copy(bet_hbm, b_v)

  gs = [g_v[pl.ds(k * LANES, LANES)] for k in range(nk)]
  bs = [b_v[pl.ds(k * LANES, LANES)] for k in range(nk)]

  inv_d = 1.0 / D
  base = wid * (n_chunks * CH)

  def chunk_body(j, carry):
    pltpu.async_copy(word_hbm.at[ids_v.at[j]], bufA, semA)
    pltpu.async_copy(segemb_hbm.at[segs_v.at[j]], bufB, semB)
    pltpu.make_async_copy(word_hbm.at[ids_v.at[j]], bufA, semA).wait()
    pltpu.make_async_copy(segemb_hbm.at[segs_v.at[j]], bufB, semB).wait()

    p0 = lax.rem(j * CH, L)

    def row_body(t, rc):
      p = p0 + t
      p = jnp.where(p >= L, p - L, p)
      xs = []
      for k in range(nk):
        sl = pl.ds(k * LANES, LANES)
        xs.append(bufA[t, sl] + bufB[t, sl] + pos_v[p, sl])
      s = xs[0]
      sq = xs[0] * xs[0]
      for k in range(1, nk):
        s = s + xs[k]
        sq = sq + xs[k] * xs[k]
      tot = jnp.sum(s)
      tot2 = jnp.sum(sq)
      mv = jnp.full((LANES,), tot, dtype=jnp.float32) * inv_d
      varv = jnp.full((LANES,), tot2, dtype=jnp.float32) * inv_d - mv * mv
      vv = varv + EPS
      ii = plsc.bitcast(vv, jnp.int32)
      ii = jnp.int32(0x5F3759DF) - (ii >> 1)
      y = plsc.bitcast(ii, jnp.float32)
      for _ in range(3):
        y = y * (1.5 - 0.5 * vv * y * y)
      for k in range(nk):
        sl = pl.ds(k * LANES, LANES)
        bufA[t, sl] = (xs[k] - mv) * y * gs[k] + bs[k]
      return rc

    lax.fori_loop(0, CH, row_body, 0, unroll=False)
    pltpu.sync_copy(bufA, out_hbm.at[pl.ds(base + j * CH, CH)])
    return carry

  lax.fori_loop(0, n_chunks, chunk_body, 0, unroll=False)


def kernel(input_ids, segment_ids, word_embedding, position_embedding,
           segment_embedding, ln_gamma, ln_beta):
  B, L = input_ids.shape
  V, D = word_embedding.shape
  total = B * L
  n_chunks = total // (NW * CH)

  ids = input_ids.reshape(NW, n_chunks, CH)
  segs = segment_ids.reshape(NW, n_chunks, CH)
  pos = position_embedding[:L]

  mesh = plsc.VectorSubcoreMesh(core_axis_name="c", subcore_axis_name="s",
                                num_cores=NC, num_subcores=NS)
  body = functools.partial(_emb_body, n_chunks=n_chunks, L=L, D=D)
  run = pl.kernel(
      body,
      out_type=jax.ShapeDtypeStruct((total, D), jnp.float32),
      mesh=mesh,
      compiler_params=pltpu.CompilerParams(needs_layout_passes=False),
      scratch_types=[
          pltpu.VMEM((n_chunks, CH), jnp.int32),
          pltpu.VMEM((n_chunks, CH), jnp.int32),
          pltpu.VMEM((L, D), jnp.float32),
          pltpu.VMEM((D,), jnp.float32),
          pltpu.VMEM((D,), jnp.float32),
          pltpu.VMEM((CH, D), jnp.float32),
          pltpu.VMEM((CH, D), jnp.float32),
          pltpu.SemaphoreType.DMA,
          pltpu.SemaphoreType.DMA,
      ],
  )
  out = run(ids, segs, word_embedding, pos, segment_embedding,
            ln_gamma, ln_beta)
  return out.reshape(B, L, D)
